# Initial kernel scaffold; baseline (speedup 1.0000x reference)
import jax
import jax.numpy as jnp
from jax.experimental import pallas as pl


def kernel(edge_index, feat, unsplice, splice, alpha0, beta0, gamma0, dt, W1, b1, al1, ar1, W2, b2, al2, ar2, Wl, bl):
    raise NotImplementedError("write your pallas kernel here")


# speedup vs baseline: 17.6833x; 17.6833x over previous
import functools

import jax
import jax.numpy as jnp
from jax import lax
from jax.experimental import pallas as pl
from jax.experimental.pallas import tpu as pltpu
from jax.experimental.pallas import tpu_sc as plsc

N = 10000
E = 160000
IN_FEATS = 128
H1 = 128
H2 = 64

NC = 2
NS = 16
NW = NC * NS
LANES = 16

N_PAD = 10240
R = 512
K = 32
CPT = (E + NW * K - 1) // (NW * K)
EPT = CPT * K
E_PAD = EPT * NW

F1E = 144
F2E = 80


def _elu(x):
    return jnp.where(x > 0, x, jnp.exp(jnp.minimum(x, 0.0)) - 1.0)


def _tc_pre_body(f_ref, w_ref, alr_ref, hx_ref, eler_ref, mm_ref):
    i = pl.program_id(0)
    fext = hx_ref.shape[1]
    ones_col = fext - LANES
    h = lax.dot_general(f_ref[...], w_ref[...], (((1,), (1,)), ((), ())),
                        preferred_element_type=jnp.float32)
    lane = lax.broadcasted_iota(jnp.int32, h.shape, 1)
    h = h + jnp.where(lane == ones_col, 1.0, 0.0)
    hx_ref[...] = h
    eler = lax.dot_general(alr_ref[...], h, (((1,), (1,)), ((), ())),
                           preferred_element_type=jnp.float32)
    eler_ref[...] = eler
    mblk = jnp.max(eler, axis=1, keepdims=True)

    @pl.when(i == 0)
    def _():
        mm_ref[...] = mblk

    @pl.when(i > 0)
    def _():
        mm_ref[...] = jnp.maximum(mm_ref[...], mblk)


def _tc_pre(feat_p, w1p, alr1):
    grid = (N_PAD // R,)
    return pl.pallas_call(
        _tc_pre_body,
        grid=grid,
        in_specs=[
            pl.BlockSpec((R, IN_FEATS), lambda i: (i, 0)),
            pl.BlockSpec((F1E, IN_FEATS), lambda i: (0, 0)),
            pl.BlockSpec((2, F1E), lambda i: (0, 0)),
        ],
        out_specs=[
            pl.BlockSpec((R, F1E), lambda i: (i, 0)),
            pl.BlockSpec((2, R), lambda i: (0, i)),
            pl.BlockSpec((2, 1), lambda i: (0, 0)),
        ],
        out_shape=[
            jax.ShapeDtypeStruct((N_PAD, F1E), jnp.float32),
            jax.ShapeDtypeStruct((2, N_PAD), jnp.float32),
            jax.ShapeDtypeStruct((2, 1), jnp.float32),
        ],
    )(feat_p, w1p, alr1)


def _tc_mid_body(ua_ref, ub_ref, b1_ref, w2_ref, alr_ref, hx_ref, eler_ref, mm_ref):
    i = pl.program_id(0)
    u = ua_ref[...] + ub_ref[...]
    denom = jnp.maximum(u[:, H1:H1 + 1], 1e-9)
    rst = u[:, :H1] / denom + b1_ref[...]
    x = _elu(_elu(rst))
    h = lax.dot_general(x, w2_ref[...], (((1,), (1,)), ((), ())),
                        preferred_element_type=jnp.float32)
    lane = lax.broadcasted_iota(jnp.int32, h.shape, 1)
    h = h + jnp.where(lane == H2, 1.0, 0.0)
    hx_ref[...] = h
    eler = lax.dot_general(alr_ref[...], h, (((1,), (1,)), ((), ())),
                           preferred_element_type=jnp.float32)
    eler_ref[...] = eler
    mblk = jnp.max(eler, axis=1, keepdims=True)

    @pl.when(i == 0)
    def _():
        mm_ref[...] = mblk

    @pl.when(i > 0)
    def _():
        mm_ref[...] = jnp.maximum(mm_ref[...], mblk)


def _tc_mid(ua, ub, b1, w2p, alr2):
    grid = (N_PAD // R,)
    return pl.pallas_call(
        _tc_mid_body,
        grid=grid,
        in_specs=[
            pl.BlockSpec((R, F1E), lambda i: (i, 0)),
            pl.BlockSpec((R, F1E), lambda i: (i, 0)),
            pl.BlockSpec((1, H1), lambda i: (0, 0)),
            pl.BlockSpec((F2E, H1), lambda i: (0, 0)),
            pl.BlockSpec((2, F2E), lambda i: (0, 0)),
        ],
        out_specs=[
            pl.BlockSpec((R, F2E), lambda i: (i, 0)),
            pl.BlockSpec((2, R), lambda i: (0, i)),
            pl.BlockSpec((2, 1), lambda i: (0, 0)),
        ],
        out_shape=[
            jax.ShapeDtypeStruct((N_PAD, F2E), jnp.float32),
            jax.ShapeDtypeStruct((2, N_PAD), jnp.float32),
            jax.ShapeDtypeStruct((2, 1), jnp.float32),
        ],
    )(ua, ub, b1, w2p, alr2)


def _tc_final_body(ua_ref, ub_ref, b2_ref, wl_ref, bl_ref, us_ref, sp_ref,
                   scal_ref, out_ref):
    u = ua_ref[...] + ub_ref[...]
    denom = jnp.maximum(u[:, H2:H2 + 1], 1e-9)
    x = _elu(u[:, :H2] / denom + b2_ref[...])
    zt = lax.dot_general(wl_ref[...], x, (((1,), (1,)), ((), ())),
                         preferred_element_type=jnp.float32)
    zt = zt + bl_ref[...]
    sig = 1.0 / (1.0 + jnp.exp(-zt))
    alpha0 = scal_ref[0:1, 0:1]
    beta0 = scal_ref[0:1, 1:2]
    gamma0 = scal_ref[0:1, 2:3]
    dt = scal_ref[0:1, 3:4]
    beta = sig[0:1, :] * beta0
    gamma = sig[1:2, :] * gamma0
    alphas = sig[2:3, :] * alpha0
    us = us_ref[...]
    sp = sp_ref[...]
    up_out = us + (alphas - beta * us) * dt
    sp_out = sp + (beta * us - gamma * sp) * dt
    zero3 = jnp.zeros((3, up_out.shape[1]), jnp.float32)
    out_ref[...] = jnp.concatenate([up_out, sp_out, alphas, beta, gamma, zero3], 0)


def _tc_final(ua, ub, b2, wlp, blp, us, sp, scal):
    grid = (N_PAD // R,)
    return pl.pallas_call(
        _tc_final_body,
        grid=grid,
        in_specs=[
            pl.BlockSpec((R, F2E), lambda i: (i, 0)),
            pl.BlockSpec((R, F2E), lambda i: (i, 0)),
            pl.BlockSpec((1, H2), lambda i: (0, 0)),
            pl.BlockSpec((8, H2), lambda i: (0, 0)),
            pl.BlockSpec((8, 1), lambda i: (0, 0)),
            pl.BlockSpec((1, R), lambda i: (0, i)),
            pl.BlockSpec((1, R), lambda i: (0, i)),
            pl.BlockSpec((1, 4), lambda i: (0, 0)),
        ],
        out_specs=[pl.BlockSpec((8, R), lambda i: (0, i))],
        out_shape=[jax.ShapeDtypeStruct((8, N_PAD), jnp.float32)],
    )(ua, ub, b2, wlp, blp, us, sp, scal)


ZROWS = 16


def _make_sc_edge(fext):
    mesh = plsc.VectorSubcoreMesh(core_axis_name="c", subcore_axis_name="s")
    rpt = N_PAD // NS

    def body(hx_hbm, eler_hbm, mvec_hbm, srcp_hbm, dstp_hbm, out_hbm,
             el_v, er_v, src_v, dst_v, m_v, ee_v, rows_v, zero_v, u_sh, sem):
        cid = lax.axis_index("c")
        sid = lax.axis_index("s")
        wid = sid * NC + cid

        pltpu.sync_copy(eler_hbm.at[0], el_v)
        pltpu.sync_copy(eler_hbm.at[1], er_v)
        pltpu.sync_copy(mvec_hbm, m_v)
        pltpu.sync_copy(srcp_hbm.at[wid], src_v)
        pltpu.sync_copy(dstp_hbm.at[wid], dst_v)

        zrow = jnp.zeros((LANES,), jnp.float32)
        def zb_row(zi, _):
            def zcol(qi, _):
                zero_v[zi, pl.ds(qi * LANES, LANES)] = zrow
                return 0
            lax.fori_loop(0, fext // LANES, zcol, 0)
            return 0
        lax.fori_loop(0, ZROWS, zb_row, 0)

        def zdma(ji, _):
            pltpu.sync_copy(zero_v, u_sh.at[pl.ds(sid * rpt + ji * ZROWS, ZROWS)])
            return 0
        lax.fori_loop(0, rpt // ZROWS, zdma, 0)
        plsc.subcore_barrier()

        mvec = m_v[...]

        def chunk(ci, _):
            for j in range(K // LANES):
                sv = src_v[ci, pl.ds(j * LANES, LANES)]
                dv = dst_v[ci, pl.ds(j * LANES, LANES)]
                elg = plsc.load_gather(el_v, [sv])
                erg = plsc.load_gather(er_v, [dv])
                x = elg + erg
                e = jnp.where(x >= 0, x, x * 0.2)
                ee_v[pl.ds(j * LANES, LANES)] = jnp.exp(e - mvec)
            pltpu.async_copy(hx_hbm.at[src_v.at[ci]], rows_v, sem).wait()

            def scale(ri, _):
                s = ee_v[pl.ds(ri, LANES)][0]
                for q in range(fext // LANES):
                    rows_v[ri, pl.ds(q * LANES, LANES)] = (
                        rows_v[ri, pl.ds(q * LANES, LANES)] * s)
                return 0
            lax.fori_loop(0, K, scale, 0)
            pltpu.sync_copy(rows_v, u_sh.at[dst_v.at[ci]], add=True)
            return 0
        lax.fori_loop(0, CPT, chunk, 0)

        plsc.subcore_barrier()
        pltpu.sync_copy(u_sh.at[pl.ds(sid * rpt, rpt)],
                        out_hbm.at[cid, pl.ds(sid * rpt, rpt)])

    return functools.partial(
        pl.kernel,
        out_type=[jax.ShapeDtypeStruct((NC, N_PAD, fext), jnp.float32)],
        mesh=mesh,
        compiler_params=pltpu.CompilerParams(
            needs_layout_passes=False, use_tc_tiling_on_sc=False),
        scratch_types=[
            pltpu.VMEM((N_PAD,), jnp.float32),
            pltpu.VMEM((N_PAD,), jnp.float32),
            pltpu.VMEM((CPT, K), jnp.int32),
            pltpu.VMEM((CPT, K), jnp.int32),
            pltpu.VMEM((LANES,), jnp.float32),
            pltpu.VMEM((K + LANES,), jnp.float32),
            pltpu.VMEM((K, fext), jnp.float32),
            pltpu.VMEM((ZROWS, fext), jnp.float32),
            pltpu.VMEM_SHARED((N_PAD, fext), jnp.float32),
            pltpu.SemaphoreType.DMA,
        ],
    )(body)


_sc_edge_1 = _make_sc_edge(F1E)
_sc_edge_2 = _make_sc_edge(F2E)


def kernel(edge_index, feat, unsplice, splice, alpha0, beta0, gamma0, dt,
           W1, b1, al1, ar1, W2, b2, al2, ar2, Wl, bl):
    f32 = jnp.float32
    src = edge_index[0]
    dst = edge_index[1]
    pad_e = E_PAD - E
    srcp = jnp.concatenate([src, jnp.zeros((pad_e,), jnp.int32)]).reshape(NW, CPT, K)
    dstp = jnp.concatenate([dst, jnp.full((pad_e,), N, jnp.int32)]).reshape(NW, CPT, K)

    feat_p = jnp.zeros((N_PAD, IN_FEATS), f32).at[:N].set(feat)
    w1p = jnp.zeros((F1E, IN_FEATS), f32).at[:H1].set(W1)
    alr1 = jnp.zeros((2, F1E), f32).at[0, :H1].set(al1[0]).at[1, :H1].set(ar1[0])
    w2p = jnp.zeros((F2E, H1), f32).at[:H2].set(W2)
    alr2 = jnp.zeros((2, F2E), f32).at[0, :H2].set(al2[0]).at[1, :H2].set(ar2[0])
    wlp = jnp.zeros((8, H2), f32).at[:3].set(Wl)
    blp = jnp.zeros((8, 1), f32).at[:3, 0].set(bl)
    usp = jnp.zeros((1, N_PAD), f32).at[0, :N].set(unsplice)
    spp = jnp.zeros((1, N_PAD), f32).at[0, :N].set(splice)
    scal = jnp.stack([alpha0[0], beta0[0], gamma0[0], dt[0]]).reshape(1, 4)

    hx1, eler1, mm1 = _tc_pre(feat_p, w1p, alr1)
    mvec1 = jnp.full((LANES,), mm1[0, 0] + mm1[1, 0], f32)
    (u1,) = _sc_edge_1(hx1, eler1, mvec1, srcp, dstp)

    hx2, eler2, mm2 = _tc_mid(u1[0], u1[1], b1.reshape(1, H1), w2p, alr2)
    mvec2 = jnp.full((LANES,), mm2[0, 0] + mm2[1, 0], f32)
    (u2,) = _sc_edge_2(hx2, eler2, mvec2, srcp, dstp)

    (out8,) = _tc_final(u2[0], u2[1], b2.reshape(1, H2), wlp, blp, usp, spp, scal)

    return (out8[0, :N], out8[1, :N], out8[2, :N], out8[3, :N], out8[4, :N])
